# baseline (speedup 1.0000x reference)
import functools

import jax
import jax.numpy as jnp
from jax import lax
from jax.experimental import pallas as pl
from jax.experimental.pallas import tpu as pltpu
from jax.experimental.pallas import tpu_sc as plsc

BS = 64
NQ = 1024
L = 16
NC = 2
NS = 16
CHUNKS = NQ // L


def _matcher_body(coords_hbm, bcx_hbm, bcy_hbm, bw_hbm, bh_hbm,
                  idx_hbm, cnt_hbm, delta_hbm,
                  boxv, crowA, crowB, irowA, irowB, drowA, drowB,
                  cvecv, semA, semB, semX, semO):
    cid = lax.axis_index("c")
    sid = lax.axis_index("s")
    rA = cid * 32 + sid * 2
    rB = rA + 1
    iota = lax.iota(jnp.int32, L)
    nq_v = jnp.full((L,), NQ, jnp.int32)
    one_v = jnp.full((L,), 1, jnp.int32)
    zero_v = jnp.zeros((L,), jnp.int32)
    half_v = jnp.full((L,), 0.5, jnp.float32)
    c127_v = jnp.full((L,), 127, jnp.int32)

    hA = pltpu.async_copy(coords_hbm.at[rA], crowA, semA)
    hB = pltpu.async_copy(coords_hbm.at[rB], crowB, semB)
    h0 = pltpu.async_copy(bcx_hbm, boxv.at[pl.ds(0, BS)], semX)
    h1 = pltpu.async_copy(bcy_hbm, boxv.at[pl.ds(BS, BS)], semX)
    h2 = pltpu.async_copy(bw_hbm, boxv.at[pl.ds(2 * BS, BS)], semX)
    h3 = pltpu.async_copy(bh_hbm, boxv.at[pl.ds(3 * BS, BS)], semX)

    @plsc.parallel_loop(0, CHUNKS, unroll=2)
    def _fill(j):
        row = zero_v + lax.shift_right_logical(j, 3)
        col = (j & 7) * L + iota
        plsc.store_scatter(irowA, [row, col], nq_v)
        plsc.store_scatter(irowB, [row, col], nq_v)

    h0.wait()
    h1.wait()
    h2.wait()
    h3.wait()

    def box_vecs(r):
        base = zero_v + r
        cx = plsc.load_gather(boxv, [base])
        cy = plsc.load_gather(boxv, [base + BS])
        w = plsc.load_gather(boxv, [base + 2 * BS])
        h = plsc.load_gather(boxv, [base + 3 * BS])
        return (cx, cy, cx - w * half_v, cx + w * half_v,
                cy - h * half_v, cy + h * half_v)

    cxA, cyA, xminA, xmaxA, yminA, ymaxA = box_vecs(rA)
    cxB, cyB, xminB, xmaxB, yminB, ymaxB = box_vecs(rB)
    hA.wait()
    hB.wait()

    @plsc.parallel_loop(0, CHUNKS, unroll=2, carry=(zero_v, zero_v))
    def _compact(j, carry):
        cA, cB = carry
        qi = j * L + iota
        srow = zero_v + lax.shift_right_logical(j, 3) * 2
        scol = (j & 7) * L + iota
        xA = plsc.load_gather(crowA, [srow, scol])
        yA = plsc.load_gather(crowA, [srow + 1, scol])
        xB = plsc.load_gather(crowB, [srow, scol])
        yB = plsc.load_gather(crowB, [srow + 1, scol])
        mA = (((xA - xminA) * (xA - xmaxA) < 0.0)
              & ((yA - yminA) * (yA - ymaxA) < 0.0))
        mB = (((xB - xminB) * (xB - xmaxB) < 0.0)
              & ((yB - yminB) * (yB - ymaxB) < 0.0))
        plsc.store_scatter(drowA, [srow, scol], cxA - xA)
        plsc.store_scatter(drowA, [srow + 1, scol], cyA - yA)
        plsc.store_scatter(drowB, [srow, scol], cxB - xB)
        plsc.store_scatter(drowB, [srow + 1, scol], cyB - yB)
        pA = plsc.cumsum(jnp.where(mA, one_v, zero_v))
        pB = plsc.cumsum(jnp.where(mB, one_v, zero_v))
        posA = cA + pA - one_v
        posB = cB + pB - one_v
        plsc.store_scatter(
            irowA, [lax.shift_right_logical(posA, 7), posA & c127_v],
            qi, mask=mA)
        plsc.store_scatter(
            irowB, [lax.shift_right_logical(posB, 7), posB & c127_v],
            qi, mask=mB)
        return (cA + plsc.all_reduce_population_count(mA),
                cB + plsc.all_reduce_population_count(mB))

    cA_fin, cB_fin = _compact

    hoA = pltpu.async_copy(irowA, idx_hbm.at[lax.shift_right_logical(rA, 3),
                                             :, rA & 7], semO)
    hoB = pltpu.async_copy(irowB, idx_hbm.at[lax.shift_right_logical(rB, 3),
                                             :, rB & 7], semO)
    hdA = pltpu.async_copy(drowA, delta_hbm.at[rA], semO)
    hdB = pltpu.async_copy(drowB, delta_hbm.at[rB], semO)

    cvecv[...] = jnp.where(iota == 0, cA_fin, cB_fin)
    pltpu.sync_copy(cvecv, cnt_hbm.at[cid * NS + sid])

    hoA.wait()
    hoB.wait()
    hdA.wait()
    hdB.wait()


_matcher = functools.partial(
    pl.kernel,
    mesh=plsc.VectorSubcoreMesh(core_axis_name="c", subcore_axis_name="s"),
    compiler_params=pltpu.CompilerParams(needs_layout_passes=False),
    out_type=(
        jax.ShapeDtypeStruct((8, 8, 8, 128), jnp.int32),
        jax.ShapeDtypeStruct((2 * NS, L), jnp.int32),
        jax.ShapeDtypeStruct((BS, 16, 128), jnp.float32),
    ),
    scratch_types=[
        pltpu.VMEM((4 * BS,), jnp.float32),
        pltpu.VMEM((16, 128), jnp.float32),
        pltpu.VMEM((16, 128), jnp.float32),
        pltpu.VMEM((8, 128), jnp.int32),
        pltpu.VMEM((8, 128), jnp.int32),
        pltpu.VMEM((16, 128), jnp.float32),
        pltpu.VMEM((16, 128), jnp.float32),
        pltpu.VMEM((L,), jnp.int32),
        pltpu.SemaphoreType.DMA,
        pltpu.SemaphoreType.DMA,
        pltpu.SemaphoreType.DMA,
        pltpu.SemaphoreType.DMA,
    ],
)(_matcher_body)


def kernel(bilinear_coords, boxes):
    coords = (bilinear_coords.reshape(BS, 8, 128, 2)
              .transpose(0, 1, 3, 2).reshape(BS, 16, 128))
    idx4, cnt, delta = _matcher(coords, boxes[:, 0, 0], boxes[:, 0, 1],
                                boxes[:, 0, 2], boxes[:, 0, 3])
    padded_idx = idx4.transpose(0, 2, 1, 3).reshape(BS, NQ)
    delta_out = (delta.reshape(BS, 8, 2, 128)
                 .transpose(0, 1, 3, 2).reshape(BS, NQ, 2))
    counts = cnt[:, :2].reshape(BS)
    return (padded_idx.astype(jnp.int64), counts.astype(jnp.int64), delta_out)


# speedup vs baseline: 1.0443x; 1.0443x over previous
import functools

import jax
import jax.numpy as jnp
from jax import lax
from jax.experimental import pallas as pl
from jax.experimental.pallas import tpu as pltpu
from jax.experimental.pallas import tpu_sc as plsc

BS = 64
NQ = 1024
L = 16
NC = 2
NS = 16
CHUNKS = NQ // L


def _matcher_body(coords_hbm, bcx_hbm, bcy_hbm, bw_hbm, bh_hbm,
                  idx_hbm, cnt_hbm, delta_hbm,
                  boxv, crowA, crowB, irowA, irowB, drowA, drowB,
                  sbuf, outv, cvecv, shared, semA, semB, semX, semO):
    cid = lax.axis_index("c")
    sid = lax.axis_index("s")
    rA = cid * 32 + sid * 2
    rB = rA + 1
    iota = lax.iota(jnp.int32, L)
    nq_v = jnp.full((L,), NQ, jnp.int32)
    one_v = jnp.full((L,), 1, jnp.int32)
    zero_v = jnp.zeros((L,), jnp.int32)
    half_v = jnp.full((L,), 0.5, jnp.float32)
    c127_v = jnp.full((L,), 127, jnp.int32)

    hA = pltpu.async_copy(coords_hbm.at[rA], crowA, semA)
    hB = pltpu.async_copy(coords_hbm.at[rB], crowB, semB)
    h0 = pltpu.async_copy(bcx_hbm, boxv.at[pl.ds(0, BS)], semX)
    h1 = pltpu.async_copy(bcy_hbm, boxv.at[pl.ds(BS, BS)], semX)
    h2 = pltpu.async_copy(bw_hbm, boxv.at[pl.ds(2 * BS, BS)], semX)
    h3 = pltpu.async_copy(bh_hbm, boxv.at[pl.ds(3 * BS, BS)], semX)

    @plsc.parallel_loop(0, CHUNKS, unroll=2)
    def _fill(j):
        row = zero_v + lax.shift_right_logical(j, 3)
        col = (j & 7) * L + iota
        plsc.store_scatter(irowA, [row, col], nq_v)
        plsc.store_scatter(irowB, [row, col], nq_v)

    h0.wait()
    h1.wait()
    h2.wait()
    h3.wait()

    def box_vecs(r):
        base = zero_v + r
        cx = plsc.load_gather(boxv, [base])
        cy = plsc.load_gather(boxv, [base + BS])
        w = plsc.load_gather(boxv, [base + 2 * BS])
        h = plsc.load_gather(boxv, [base + 3 * BS])
        return (cx, cy, cx - w * half_v, cx + w * half_v,
                cy - h * half_v, cy + h * half_v)

    cxA, cyA, xminA, xmaxA, yminA, ymaxA = box_vecs(rA)
    cxB, cyB, xminB, xmaxB, yminB, ymaxB = box_vecs(rB)
    hA.wait()
    hB.wait()

    @plsc.parallel_loop(0, CHUNKS, unroll=2, carry=(zero_v, zero_v))
    def _compact(j, carry):
        cA, cB = carry
        qi = j * L + iota
        srow = zero_v + lax.shift_right_logical(j, 3) * 2
        scol = (j & 7) * L + iota
        xA = plsc.load_gather(crowA, [srow, scol])
        yA = plsc.load_gather(crowA, [srow + 1, scol])
        xB = plsc.load_gather(crowB, [srow, scol])
        yB = plsc.load_gather(crowB, [srow + 1, scol])
        mA = (((xA - xminA) * (xA - xmaxA) < 0.0)
              & ((yA - yminA) * (yA - ymaxA) < 0.0))
        mB = (((xB - xminB) * (xB - xmaxB) < 0.0)
              & ((yB - yminB) * (yB - ymaxB) < 0.0))
        plsc.store_scatter(drowA, [srow, scol], cxA - xA)
        plsc.store_scatter(drowA, [srow + 1, scol], cyA - yA)
        plsc.store_scatter(drowB, [srow, scol], cxB - xB)
        plsc.store_scatter(drowB, [srow + 1, scol], cyB - yB)
        pA = plsc.cumsum(jnp.where(mA, one_v, zero_v))
        pB = plsc.cumsum(jnp.where(mB, one_v, zero_v))
        posA = cA + pA - one_v
        posB = cB + pB - one_v
        plsc.store_scatter(
            irowA, [lax.shift_right_logical(posA, 7), posA & c127_v],
            qi, mask=mA)
        plsc.store_scatter(
            irowB, [lax.shift_right_logical(posB, 7), posB & c127_v],
            qi, mask=mB)
        return (cA + plsc.all_reduce_population_count(mA),
                cB + plsc.all_reduce_population_count(mB))

    cA_fin, cB_fin = _compact

    hoA = pltpu.async_copy(irowA, idx_hbm.at[lax.shift_right_logical(rA, 3),
                                             :, rA & 7], semO)
    hoB = pltpu.async_copy(irowB, idx_hbm.at[lax.shift_right_logical(rB, 3),
                                             :, rB & 7], semO)
    hdA = pltpu.async_copy(drowA, delta_hbm.at[rA], semO)
    hdB = pltpu.async_copy(drowB, delta_hbm.at[rB], semO)

    cvecv[...] = jnp.where(iota == 0, cA_fin, cB_fin)
    pltpu.sync_copy(cvecv, shared.at[pl.ds(sid * L, L)])
    plsc.subcore_barrier()

    @pl.when(sid == 0)
    def _assemble():
        pltpu.sync_copy(shared, sbuf)
        for t in range(2):
            iv = t * L + iota
            flat = lax.shift_right_logical(iv, 1) * L + lax.bitwise_and(iv, one_v)
            outv[pl.ds(t * L, L)] = plsc.load_gather(sbuf, [flat])
        pltpu.sync_copy(outv, cnt_hbm.at[pl.ds(cid * 32, 32)])

    hoA.wait()
    hoB.wait()
    hdA.wait()
    hdB.wait()


_matcher = functools.partial(
    pl.kernel,
    mesh=plsc.VectorSubcoreMesh(core_axis_name="c", subcore_axis_name="s"),
    compiler_params=pltpu.CompilerParams(needs_layout_passes=False),
    out_type=(
        jax.ShapeDtypeStruct((8, 8, 8, 128), jnp.int32),
        jax.ShapeDtypeStruct((BS,), jnp.int32),
        jax.ShapeDtypeStruct((BS, 16, 128), jnp.float32),
    ),
    scratch_types=[
        pltpu.VMEM((4 * BS,), jnp.float32),
        pltpu.VMEM((16, 128), jnp.float32),
        pltpu.VMEM((16, 128), jnp.float32),
        pltpu.VMEM((8, 128), jnp.int32),
        pltpu.VMEM((8, 128), jnp.int32),
        pltpu.VMEM((16, 128), jnp.float32),
        pltpu.VMEM((16, 128), jnp.float32),
        pltpu.VMEM((NS * L,), jnp.int32),
        pltpu.VMEM((2 * NS,), jnp.int32),
        pltpu.VMEM((L,), jnp.int32),
        pltpu.VMEM_SHARED((NS * L,), jnp.int32),
        pltpu.SemaphoreType.DMA,
        pltpu.SemaphoreType.DMA,
        pltpu.SemaphoreType.DMA,
        pltpu.SemaphoreType.DMA,
    ],
)(_matcher_body)


def kernel(bilinear_coords, boxes):
    coords = (bilinear_coords.reshape(BS, 8, 128, 2)
              .transpose(0, 1, 3, 2).reshape(BS, 16, 128))
    idx4, cnt, delta = _matcher(coords, boxes[:, 0, 0], boxes[:, 0, 1],
                                boxes[:, 0, 2], boxes[:, 0, 3])
    padded_idx = idx4.transpose(0, 2, 1, 3).reshape(BS, NQ)
    delta_out = (delta.reshape(BS, 8, 2, 128)
                 .transpose(0, 1, 3, 2).reshape(BS, NQ, 2))
    return (padded_idx.astype(jnp.int64), cnt.astype(jnp.int64), delta_out)


# speedup vs baseline: 1.0486x; 1.0041x over previous
import functools

import jax
import jax.numpy as jnp
from jax import lax
from jax.experimental import pallas as pl
from jax.experimental.pallas import tpu as pltpu
from jax.experimental.pallas import tpu_sc as plsc

BS = 64
NQ = 1024
L = 16
NC = 2
NS = 16
CHUNKS = NQ // L


def _matcher_body(coords_hbm, bcx_hbm, bcy_hbm, bw_hbm, bh_hbm,
                  idx_hbm, cnt_hbm, delta_hbm,
                  boxv, crowA, crowB, irowA, irowB, drowA, drowB,
                  sbuf, outv, cvecv, shared, semA, semB, semX, semO):
    cid = lax.axis_index("c")
    sid = lax.axis_index("s")
    rA = cid * 32 + sid * 2
    rB = rA + 1
    iota = lax.iota(jnp.int32, L)
    nq_v = jnp.full((L,), NQ, jnp.int32)
    one_v = jnp.full((L,), 1, jnp.int32)
    zero_v = jnp.zeros((L,), jnp.int32)
    half_v = jnp.full((L,), 0.5, jnp.float32)
    c127_v = jnp.full((L,), 127, jnp.int32)

    hA = pltpu.async_copy(coords_hbm.at[rA], crowA, semA)
    hB = pltpu.async_copy(coords_hbm.at[rB], crowB, semB)
    h0 = pltpu.async_copy(bcx_hbm, boxv.at[pl.ds(0, BS)], semX)
    h1 = pltpu.async_copy(bcy_hbm, boxv.at[pl.ds(BS, BS)], semX)
    h2 = pltpu.async_copy(bw_hbm, boxv.at[pl.ds(2 * BS, BS)], semX)
    h3 = pltpu.async_copy(bh_hbm, boxv.at[pl.ds(3 * BS, BS)], semX)

    h0.wait()
    h1.wait()
    h2.wait()
    h3.wait()

    def box_vecs(r):
        base = zero_v + r
        cx = plsc.load_gather(boxv, [base])
        cy = plsc.load_gather(boxv, [base + BS])
        w = plsc.load_gather(boxv, [base + 2 * BS])
        h = plsc.load_gather(boxv, [base + 3 * BS])
        return (cx, cy, cx - w * half_v, cx + w * half_v,
                cy - h * half_v, cy + h * half_v)

    cxA, cyA, xminA, xmaxA, yminA, ymaxA = box_vecs(rA)
    cxB, cyB, xminB, xmaxB, yminB, ymaxB = box_vecs(rB)
    hA.wait()
    hB.wait()

    @plsc.parallel_loop(0, CHUNKS, unroll=2, carry=(zero_v, zero_v))
    def _compact(j, carry):
        cA, cB = carry
        qi = j * L + iota
        frow = zero_v + lax.shift_right_logical(j, 3)
        srow = frow * 2
        scol = (j & 7) * L + iota
        plsc.store_scatter(irowA, [frow, scol], nq_v)
        plsc.store_scatter(irowB, [frow, scol], nq_v)
        xA = plsc.load_gather(crowA, [srow, scol])
        yA = plsc.load_gather(crowA, [srow + 1, scol])
        xB = plsc.load_gather(crowB, [srow, scol])
        yB = plsc.load_gather(crowB, [srow + 1, scol])
        mA = (((xA - xminA) * (xA - xmaxA) < 0.0)
              & ((yA - yminA) * (yA - ymaxA) < 0.0))
        mB = (((xB - xminB) * (xB - xmaxB) < 0.0)
              & ((yB - yminB) * (yB - ymaxB) < 0.0))
        plsc.store_scatter(drowA, [srow, scol], cxA - xA)
        plsc.store_scatter(drowA, [srow + 1, scol], cyA - yA)
        plsc.store_scatter(drowB, [srow, scol], cxB - xB)
        plsc.store_scatter(drowB, [srow + 1, scol], cyB - yB)
        pA = plsc.cumsum(jnp.where(mA, one_v, zero_v))
        pB = plsc.cumsum(jnp.where(mB, one_v, zero_v))
        posA = cA + pA - one_v
        posB = cB + pB - one_v
        plsc.store_scatter(
            irowA, [lax.shift_right_logical(posA, 7), posA & c127_v],
            qi, mask=mA)
        plsc.store_scatter(
            irowB, [lax.shift_right_logical(posB, 7), posB & c127_v],
            qi, mask=mB)
        return (cA + plsc.all_reduce_population_count(mA),
                cB + plsc.all_reduce_population_count(mB))

    cA_fin, cB_fin = _compact

    hoA = pltpu.async_copy(irowA, idx_hbm.at[lax.shift_right_logical(rA, 3),
                                             :, rA & 7], semO)
    hoB = pltpu.async_copy(irowB, idx_hbm.at[lax.shift_right_logical(rB, 3),
                                             :, rB & 7], semO)
    hdA = pltpu.async_copy(drowA, delta_hbm.at[rA], semO)
    hdB = pltpu.async_copy(drowB, delta_hbm.at[rB], semO)

    cvecv[...] = jnp.where(iota == 0, cA_fin, cB_fin)
    pltpu.sync_copy(cvecv, shared.at[pl.ds(sid * L, L)])
    plsc.subcore_barrier()

    @pl.when(sid == 0)
    def _assemble():
        pltpu.sync_copy(shared, sbuf)
        for t in range(2):
            iv = t * L + iota
            flat = lax.shift_right_logical(iv, 1) * L + lax.bitwise_and(iv, one_v)
            outv[pl.ds(t * L, L)] = plsc.load_gather(sbuf, [flat])
        pltpu.sync_copy(outv, cnt_hbm.at[pl.ds(cid * 32, 32)])

    hoA.wait()
    hoB.wait()
    hdA.wait()
    hdB.wait()


_matcher = functools.partial(
    pl.kernel,
    mesh=plsc.VectorSubcoreMesh(core_axis_name="c", subcore_axis_name="s"),
    compiler_params=pltpu.CompilerParams(needs_layout_passes=False),
    out_type=(
        jax.ShapeDtypeStruct((8, 8, 8, 128), jnp.int32),
        jax.ShapeDtypeStruct((BS,), jnp.int32),
        jax.ShapeDtypeStruct((BS, 16, 128), jnp.float32),
    ),
    scratch_types=[
        pltpu.VMEM((4 * BS,), jnp.float32),
        pltpu.VMEM((16, 128), jnp.float32),
        pltpu.VMEM((16, 128), jnp.float32),
        pltpu.VMEM((8, 128), jnp.int32),
        pltpu.VMEM((8, 128), jnp.int32),
        pltpu.VMEM((16, 128), jnp.float32),
        pltpu.VMEM((16, 128), jnp.float32),
        pltpu.VMEM((NS * L,), jnp.int32),
        pltpu.VMEM((2 * NS,), jnp.int32),
        pltpu.VMEM((L,), jnp.int32),
        pltpu.VMEM_SHARED((NS * L,), jnp.int32),
        pltpu.SemaphoreType.DMA,
        pltpu.SemaphoreType.DMA,
        pltpu.SemaphoreType.DMA,
        pltpu.SemaphoreType.DMA,
    ],
)(_matcher_body)


def kernel(bilinear_coords, boxes):
    coords = (bilinear_coords.reshape(BS, 8, 128, 2)
              .transpose(0, 1, 3, 2).reshape(BS, 16, 128))
    idx4, cnt, delta = _matcher(coords, boxes[:, 0, 0], boxes[:, 0, 1],
                                boxes[:, 0, 2], boxes[:, 0, 3])
    padded_idx = idx4.transpose(0, 2, 1, 3).reshape(BS, NQ)
    delta_out = (delta.reshape(BS, 8, 2, 128)
                 .transpose(0, 1, 3, 2).reshape(BS, NQ, 2))
    return (padded_idx.astype(jnp.int64), cnt.astype(jnp.int64), delta_out)
